# baseline (speedup 1.0000x reference)
import functools
import numpy as np

import jax
import jax.numpy as jnp
from jax import lax
from jax.experimental import pallas as pl
from jax.experimental.pallas import tpu as pltpu
from jax.experimental.pallas import tpu_sc as plsc

_RATIO = 0.5
_DEPTH = 3
_NC = 2
_NS = 16
_NW = _NC * _NS


def _pad_to(x, m, axis=0):
    n = x.shape[axis]
    p = (-n) % m
    if p == 0:
        return x
    pads = [(0, 0)] * x.ndim
    pads[axis] = (0, p)
    return jnp.pad(x, pads)


def _pick_chunk(b_per_w, d, nbuf=1):
    cap = max(8, (400 * 1024) // (nbuf * d * 4))
    ch = b_per_w
    while ch > cap or ch % 8:
        ch -= 1
        while b_per_w % ch:
            ch -= 1
    return ch


def _sc_gather_kernel(nidx, nchunks, ch, nbuf, *refs):
    table = refs[0]
    idxs = refs[1:1 + nidx]
    outs = refs[1 + nidx:1 + 2 * nidx]
    idx_c = refs[1 + 2 * nidx:1 + 2 * nidx + nbuf]
    bufs = refs[1 + 2 * nidx + nbuf:1 + 2 * nidx + 2 * nbuf]
    isem, gsem, wsem = refs[-3], refs[-2], refs[-1]
    wid = lax.axis_index("s") * _NC + lax.axis_index("c")
    base = wid * (nchunks * ch)
    steps = [(j, c) for j in range(nidx) for c in range(nchunks)]
    T = len(steps)
    G = max(1, nbuf - 2)

    def _idx_dma(t):
        j, c = steps[t]
        return pltpu.async_copy(
            idxs[j].at[pl.ds(base + c * ch, ch)], idx_c[t % nbuf], isem)

    def _gather(t):
        return pltpu.async_copy(
            table.at[idx_c[t % nbuf]], bufs[t % nbuf], gsem)

    ih, gh, wh = {}, {}, {}
    for t in range(min(G, T)):
        ih[t] = _idx_dma(t)
    for t in range(min(G, T)):
        ih[t].wait()
        gh[t] = _gather(t)
    for t, (j, c) in enumerate(steps):
        gh[t].wait()
        wh[t] = pltpu.async_copy(
            bufs[t % nbuf], outs[j].at[pl.ds(base + c * ch, ch)], wsem)
        nxt = t + G
        if nxt < T:
            if nxt - nbuf >= 0:
                wh[nxt - nbuf].wait()
            ih[nxt] = _idx_dma(nxt)
            ih[nxt].wait()
            gh[nxt] = _gather(nxt)
    for t in range(max(0, T - nbuf), T):
        wh[t].wait()


def _sc_gather(table, idxs):
    nidx = len(idxs)
    B = idxs[0].shape[0]
    T, D = table.shape
    dt = table.dtype
    assert B % _NW == 0, B
    b_per_w = B // _NW
    nbuf = 8
    ch = _pick_chunk(b_per_w, D, nbuf=nbuf)
    nchunks = b_per_w // ch
    mesh = plsc.VectorSubcoreMesh(core_axis_name="c", subcore_axis_name="s",
                                  num_cores=_NC)
    kfn = pl.kernel(
        functools.partial(_sc_gather_kernel, nidx, nchunks, ch, nbuf),
        mesh=mesh,
        out_type=[jax.ShapeDtypeStruct((B, D), dt)] * nidx,
        scratch_types=[pltpu.VMEM((ch,), jnp.int32)] * nbuf
        + [pltpu.VMEM((ch, D), dt)] * nbuf
        + [pltpu.SemaphoreType.DMA, pltpu.SemaphoreType.DMA,
           pltpu.SemaphoreType.DMA],
    )
    outs = kfn(table, *idxs)
    return outs if nidx > 1 else outs[0]


def _mm_kernel(f_ref, w_ref, b_ref, o_ref, *, relu):
    acc = jnp.dot(f_ref[...], w_ref[...], preferred_element_type=jnp.float32)
    acc = acc + b_ref[...]
    if relu:
        acc = jnp.maximum(acc, 0.0)
    o_ref[...] = acc


def _mm(f, W, b, relu):
    n, K = f.shape
    H = W.shape[1]
    BN = 512
    fp = _pad_to(_pad_to(f, BN, 0), 128, 1)
    Wp = _pad_to(W, 128, 0)
    npad, Kp = fp.shape
    grid = (npad // BN,)
    out = pl.pallas_call(
        functools.partial(_mm_kernel, relu=relu),
        grid=grid,
        in_specs=[
            pl.BlockSpec((BN, Kp), lambda i: (i, 0)),
            pl.BlockSpec((Kp, H), lambda i: (0, 0)),
            pl.BlockSpec((1, H), lambda i: (0, 0)),
        ],
        out_specs=pl.BlockSpec((BN, H), lambda i: (i, 0)),
        out_shape=jax.ShapeDtypeStruct((npad, H), jnp.float32),
    )(fp, Wp, b.reshape(1, H))
    return out[:n]


def _mesh_conv(x, nbr_cols, W, b, relu):
    n, C = x.shape
    Cp = 128
    B = nbr_cols[0].shape[0]
    xt = _pad_to(_pad_to(x, Cp, 1), B, 0)
    ga, gb, gc, gd = _sc_gather(xt, nbr_cols)
    xa, xb_, xc, xd = (g[:n, :C] for g in (ga, gb, gc, gd))
    f = jnp.concatenate(
        [x, jnp.abs(xa - xc), xa + xc, jnp.abs(xb_ - xd), xb_ + xd], axis=1)
    return _mm(f, W, b, relu)


def _pool(x, nbr_cols, p):
    n = x.shape[0]
    npad = nbr_cols[0].shape[0]
    score = (x @ p) / (jnp.linalg.norm(p) + 1e-12)
    k = int(np.ceil(_RATIO * n))
    kpad = -(-k // 256) * 256
    vals, perm = jax.lax.top_k(score, k)
    perm_pad = _pad_to(perm.astype(jnp.int32), kpad)
    xpp = _sc_gather(_pad_to(x, npad, 0), [perm_pad])
    xp = xpp[:k] * jnp.tanh(vals)[:, None]
    inv = jnp.full((n,), -1, dtype=jnp.int32).at[perm].set(
        jnp.arange(k, dtype=jnp.int32))
    nbp_cols = []
    selfi = jnp.arange(k, dtype=jnp.int32)
    for c in nbr_cols:
        nb = inv[c[perm]]
        nbp = jnp.where(nb < 0, selfi, nb)
        nbp_cols.append(_pad_to(nbp, kpad))
    return xp, nbp_cols, perm, inv


def _unpool(x_small, inv, skip, npad):
    n = skip.shape[0]
    inv_pad = _pad_to(jnp.maximum(inv, 0), npad)
    g = _sc_gather(_pad_to(x_small, -(-x_small.shape[0] // 8) * 8, 0),
                   [inv_pad])
    return skip + jnp.where((inv >= 0)[:, None], g[:n], 0.0)


def kernel(x, edge_index, W_in, b_in, W_d1, b_d1, p1, W_d2, b_d2, p2,
           W_d3, b_d3, p3, W_u1, b_u1, W_u2, b_u2, W_u3, b_u3):
    n0 = x.shape[0]
    np0 = -(-n0 // 256) * 256
    nbr0 = edge_index[1].reshape(-1, 4).astype(jnp.int32)
    nbr0_cols = [_pad_to(nbr0[:, j], np0) for j in range(4)]
    x = _mesh_conv(x, nbr0_cols, W_in, b_in, True)
    down = [(W_d1, b_d1, p1), (W_d2, b_d2, p2), (W_d3, b_d3, p3)]
    up = [(W_u1, b_u1), (W_u2, b_u2), (W_u3, b_u3)]
    skips, invs, res_nbrs, sizes = [], [], [nbr0_cols], []
    nbr_cols = nbr0_cols
    for (W, b, p) in down:
        skips.append(x)
        sizes.append(x.shape[0])
        x, nbr_cols, perm, inv = _pool(x, nbr_cols, p)
        invs.append(inv)
        res_nbrs.append(nbr_cols)
        x = _mesh_conv(x, nbr_cols, W, b, True)
    for j in range(_DEPTH):
        i = _DEPTH - 1 - j
        W, b = up[j]
        npad_i = res_nbrs[i][0].shape[0]
        x = _unpool(x, invs[i], skips[i], npad_i)
        x = _mesh_conv(x, res_nbrs[i], W, b, j < _DEPTH - 1)
    return x


# speedup vs baseline: 1.0616x; 1.0616x over previous
import functools
import numpy as np

import jax
import jax.numpy as jnp
from jax import lax
from jax.experimental import pallas as pl
from jax.experimental.pallas import tpu as pltpu
from jax.experimental.pallas import tpu_sc as plsc

_RATIO = 0.5
_DEPTH = 3
_NC = 2
_NS = 16
_NW = _NC * _NS


def _pad_to(x, m, axis=0):
    n = x.shape[axis]
    p = (-n) % m
    if p == 0:
        return x
    pads = [(0, 0)] * x.ndim
    pads[axis] = (0, p)
    return jnp.pad(x, pads)


def _pick_chunk(b_per_w, d, nbuf=1):
    cap = max(8, (400 * 1024) // (nbuf * d * 4))
    ch = b_per_w
    while ch > cap or ch % 8:
        ch -= 1
        while b_per_w % ch:
            ch -= 1
    return ch


def _sc_gather_kernel(nidx, nchunks, ch, nbuf, *refs):
    table = refs[0]
    idxs = refs[1:1 + nidx]
    outs = refs[1 + nidx:1 + 2 * nidx]
    idx_c = refs[1 + 2 * nidx:1 + 2 * nidx + nbuf]
    bufs = refs[1 + 2 * nidx + nbuf:1 + 2 * nidx + 2 * nbuf]
    isem, gsem, wsem = refs[-3], refs[-2], refs[-1]
    wid = lax.axis_index("s") * _NC + lax.axis_index("c")
    base = wid * (nchunks * ch)
    steps = [(j, c) for j in range(nidx) for c in range(nchunks)]
    T = len(steps)
    G = max(1, nbuf - 2)

    def _idx_dma(t):
        j, c = steps[t]
        return pltpu.async_copy(
            idxs[j].at[pl.ds(base + c * ch, ch)], idx_c[t % nbuf], isem)

    def _gather(t):
        hs = []
        buf = bufs[t % nbuf]
        iv = idx_c[t % nbuf]
        for g in range(ch // 16):
            ivec = iv[pl.ds(g * 16, 16)]
            hs.append(pltpu.async_copy(
                table.at[ivec], buf.at[pl.ds(g * 16, 16)], gsem))
        return hs

    ih, gh, wh = {}, {}, {}
    for t in range(min(G, T)):
        ih[t] = _idx_dma(t)
    for t in range(min(G, T)):
        ih[t].wait()
        gh[t] = _gather(t)
    for t, (j, c) in enumerate(steps):
        for h in gh[t]:
            h.wait()
        wh[t] = pltpu.async_copy(
            bufs[t % nbuf], outs[j].at[pl.ds(base + c * ch, ch)], wsem)
        nxt = t + G
        if nxt < T:
            if nxt - nbuf >= 0:
                wh[nxt - nbuf].wait()
            ih[nxt] = _idx_dma(nxt)
            ih[nxt].wait()
            gh[nxt] = _gather(nxt)
    for t in range(max(0, T - nbuf), T):
        wh[t].wait()


def _sc_gather(table, idxs):
    nidx = len(idxs)
    B = idxs[0].shape[0]
    T, D = table.shape
    dt = table.dtype
    assert B % _NW == 0, B
    b_per_w = B // _NW
    nbuf = 8
    ch = _pick_chunk(b_per_w, D, nbuf=nbuf)
    nchunks = b_per_w // ch
    mesh = plsc.VectorSubcoreMesh(core_axis_name="c", subcore_axis_name="s",
                                  num_cores=_NC)
    kfn = pl.kernel(
        functools.partial(_sc_gather_kernel, nidx, nchunks, ch, nbuf),
        mesh=mesh,
        out_type=[jax.ShapeDtypeStruct((B, D), dt)] * nidx,
        scratch_types=[pltpu.VMEM((ch,), jnp.int32)] * nbuf
        + [pltpu.VMEM((ch, D), dt)] * nbuf
        + [pltpu.SemaphoreType.DMA, pltpu.SemaphoreType.DMA,
           pltpu.SemaphoreType.DMA],
    )
    outs = kfn(table, *idxs)
    return outs if nidx > 1 else outs[0]


def _mm_kernel(f_ref, w_ref, b_ref, o_ref, *, relu):
    acc = jnp.dot(f_ref[...], w_ref[...], preferred_element_type=jnp.float32)
    acc = acc + b_ref[...]
    if relu:
        acc = jnp.maximum(acc, 0.0)
    o_ref[...] = acc


def _mm(f, W, b, relu):
    n, K = f.shape
    H = W.shape[1]
    BN = 512
    fp = _pad_to(_pad_to(f, BN, 0), 128, 1)
    Wp = _pad_to(W, 128, 0)
    npad, Kp = fp.shape
    grid = (npad // BN,)
    out = pl.pallas_call(
        functools.partial(_mm_kernel, relu=relu),
        grid=grid,
        in_specs=[
            pl.BlockSpec((BN, Kp), lambda i: (i, 0)),
            pl.BlockSpec((Kp, H), lambda i: (0, 0)),
            pl.BlockSpec((1, H), lambda i: (0, 0)),
        ],
        out_specs=pl.BlockSpec((BN, H), lambda i: (i, 0)),
        out_shape=jax.ShapeDtypeStruct((npad, H), jnp.float32),
    )(fp, Wp, b.reshape(1, H))
    return out[:n]


def _mesh_conv(x, nbr_cols, W, b, relu):
    n, C = x.shape
    Cp = 128
    B = nbr_cols[0].shape[0]
    xt = _pad_to(_pad_to(x, Cp, 1), B, 0)
    ga, gb, gc, gd = _sc_gather(xt, nbr_cols)
    xa, xb_, xc, xd = (g[:n, :C] for g in (ga, gb, gc, gd))
    f = jnp.concatenate(
        [x, jnp.abs(xa - xc), xa + xc, jnp.abs(xb_ - xd), xb_ + xd], axis=1)
    return _mm(f, W, b, relu)


def _pool(x, nbr_cols, p):
    n = x.shape[0]
    npad = nbr_cols[0].shape[0]
    score = (x @ p) / (jnp.linalg.norm(p) + 1e-12)
    k = int(np.ceil(_RATIO * n))
    kpad = -(-k // 256) * 256
    vals, perm = jax.lax.top_k(score, k)
    perm_pad = _pad_to(perm.astype(jnp.int32), kpad)
    xpp = _sc_gather(_pad_to(x, npad, 0), [perm_pad])
    xp = xpp[:k] * jnp.tanh(vals)[:, None]
    inv = jnp.full((n,), -1, dtype=jnp.int32).at[perm].set(
        jnp.arange(k, dtype=jnp.int32))
    nbp_cols = []
    selfi = jnp.arange(k, dtype=jnp.int32)
    for c in nbr_cols:
        nb = inv[c[perm]]
        nbp = jnp.where(nb < 0, selfi, nb)
        nbp_cols.append(_pad_to(nbp, kpad))
    return xp, nbp_cols, perm, inv


def _unpool(x_small, inv, skip, npad):
    n = skip.shape[0]
    inv_pad = _pad_to(jnp.maximum(inv, 0), npad)
    g = _sc_gather(_pad_to(x_small, -(-x_small.shape[0] // 8) * 8, 0),
                   [inv_pad])
    return skip + jnp.where((inv >= 0)[:, None], g[:n], 0.0)


def kernel(x, edge_index, W_in, b_in, W_d1, b_d1, p1, W_d2, b_d2, p2,
           W_d3, b_d3, p3, W_u1, b_u1, W_u2, b_u2, W_u3, b_u3):
    n0 = x.shape[0]
    np0 = -(-n0 // 256) * 256
    nbr0 = edge_index[1].reshape(-1, 4).astype(jnp.int32)
    nbr0_cols = [_pad_to(nbr0[:, j], np0) for j in range(4)]
    x = _mesh_conv(x, nbr0_cols, W_in, b_in, True)
    down = [(W_d1, b_d1, p1), (W_d2, b_d2, p2), (W_d3, b_d3, p3)]
    up = [(W_u1, b_u1), (W_u2, b_u2), (W_u3, b_u3)]
    skips, invs, res_nbrs, sizes = [], [], [nbr0_cols], []
    nbr_cols = nbr0_cols
    for (W, b, p) in down:
        skips.append(x)
        sizes.append(x.shape[0])
        x, nbr_cols, perm, inv = _pool(x, nbr_cols, p)
        invs.append(inv)
        res_nbrs.append(nbr_cols)
        x = _mesh_conv(x, nbr_cols, W, b, True)
    for j in range(_DEPTH):
        i = _DEPTH - 1 - j
        W, b = up[j]
        npad_i = res_nbrs[i][0].shape[0]
        x = _unpool(x, invs[i], skips[i], npad_i)
        x = _mesh_conv(x, res_nbrs[i], W, b, j < _DEPTH - 1)
    return x


# speedup vs baseline: 1.1331x; 1.0673x over previous
import functools
import numpy as np

import jax
import jax.numpy as jnp
from jax import lax
from jax.experimental import pallas as pl
from jax.experimental.pallas import tpu as pltpu
from jax.experimental.pallas import tpu_sc as plsc

_RATIO = 0.5
_DEPTH = 3
_NC = 2
_NS = 16
_NW = _NC * _NS


def _pad_to(x, m, axis=0):
    n = x.shape[axis]
    p = (-n) % m
    if p == 0:
        return x
    pads = [(0, 0)] * x.ndim
    pads[axis] = (0, p)
    return jnp.pad(x, pads)


def _pick_chunk(b_per_w, d, nbuf=1):
    cap = max(8, (400 * 1024) // (nbuf * d * 4))
    ch = b_per_w
    while ch > cap or ch % 8:
        ch -= 1
        while b_per_w % ch:
            ch -= 1
    return ch


def _sc_gather_kernel(nidx, nchunks, ch, nbuf, *refs):
    table = refs[0]
    idxs = refs[1:1 + nidx]
    outs = refs[1 + nidx:1 + 2 * nidx]
    idx_c = refs[1 + 2 * nidx:1 + 2 * nidx + nbuf]
    bufs = refs[1 + 2 * nidx + nbuf:1 + 2 * nidx + 2 * nbuf]
    isem, gsem, wsem = refs[-3], refs[-2], refs[-1]
    wid = lax.axis_index("s") * _NC + lax.axis_index("c")
    base = wid * (nchunks * ch)
    steps = [(j, c) for j in range(nidx) for c in range(nchunks)]
    T = len(steps)
    G = max(1, nbuf - 2)

    def _idx_dma(t):
        j, c = steps[t]
        return pltpu.async_copy(
            idxs[j].at[pl.ds(base + c * ch, ch)], idx_c[t % nbuf], isem)

    def _gather(t):
        return pltpu.async_copy(
            table.at[idx_c[t % nbuf]], bufs[t % nbuf], gsem)

    ih, gh, wh = {}, {}, {}
    for t in range(min(G, T)):
        ih[t] = _idx_dma(t)
    for t in range(min(G, T)):
        ih[t].wait()
        gh[t] = _gather(t)
    for t, (j, c) in enumerate(steps):
        gh[t].wait()
        wh[t] = pltpu.async_copy(
            bufs[t % nbuf], outs[j].at[pl.ds(base + c * ch, ch)], wsem)
        nxt = t + G
        if nxt < T:
            if nxt - nbuf >= 0:
                wh[nxt - nbuf].wait()
            ih[nxt] = _idx_dma(nxt)
            ih[nxt].wait()
            gh[nxt] = _gather(nxt)
    for t in range(max(0, T - nbuf), T):
        wh[t].wait()


def _sc_gather(table, idxs):
    nidx = len(idxs)
    B = idxs[0].shape[0]
    T, D = table.shape
    dt = table.dtype
    assert B % _NW == 0, B
    b_per_w = B // _NW
    nbuf = 8
    ch = _pick_chunk(b_per_w, D, nbuf=nbuf)
    nchunks = b_per_w // ch
    mesh = plsc.VectorSubcoreMesh(core_axis_name="c", subcore_axis_name="s",
                                  num_cores=_NC)
    kfn = pl.kernel(
        functools.partial(_sc_gather_kernel, nidx, nchunks, ch, nbuf),
        mesh=mesh,
        out_type=[jax.ShapeDtypeStruct((B, D), dt)] * nidx,
        scratch_types=[pltpu.VMEM((ch,), jnp.int32)] * nbuf
        + [pltpu.VMEM((ch, D), dt)] * nbuf
        + [pltpu.SemaphoreType.DMA, pltpu.SemaphoreType.DMA,
           pltpu.SemaphoreType.DMA],
    )
    outs = kfn(table, *idxs)
    return outs if nidx > 1 else outs[0]


def _conv_kernel(x_ref, ga_ref, gb_ref, gc_ref, gd_ref, w_ref, b_ref, o_ref,
                 *, relu):
    xa = ga_ref[...]
    xb = gb_ref[...]
    xc = gc_ref[...]
    xd = gd_ref[...]
    w = w_ref[...]
    acc = jnp.dot(x_ref[...], w[0], preferred_element_type=jnp.float32)
    acc += jnp.dot(jnp.abs(xa - xc), w[1], preferred_element_type=jnp.float32)
    acc += jnp.dot(xa + xc, w[2], preferred_element_type=jnp.float32)
    acc += jnp.dot(jnp.abs(xb - xd), w[3], preferred_element_type=jnp.float32)
    acc += jnp.dot(xb + xd, w[4], preferred_element_type=jnp.float32)
    acc = acc + b_ref[...]
    if relu:
        acc = jnp.maximum(acc, 0.0)
    o_ref[...] = acc


def _conv_mm(xt, ga, gb, gc, gd, W, b, relu, C, n):
    B, Cp = xt.shape
    H = W.shape[1]
    BN = 512 if B % 512 == 0 else 256
    Wb = jnp.pad(W.reshape(5, C, H), ((0, 0), (0, Cp - C), (0, 0)))
    grid = (B // BN,)
    out = pl.pallas_call(
        functools.partial(_conv_kernel, relu=relu),
        grid=grid,
        in_specs=[pl.BlockSpec((BN, Cp), lambda i: (i, 0))] * 5
        + [pl.BlockSpec((5, Cp, H), lambda i: (0, 0, 0)),
           pl.BlockSpec((1, H), lambda i: (0, 0))],
        out_specs=pl.BlockSpec((BN, H), lambda i: (i, 0)),
        out_shape=jax.ShapeDtypeStruct((B, H), jnp.float32),
    )(xt, ga, gb, gc, gd, Wb, b.reshape(1, H))
    return out[:n]


def _mesh_conv(x, nbr_cols, W, b, relu):
    n, C = x.shape
    Cp = 128
    B = nbr_cols[0].shape[0]
    xt = _pad_to(_pad_to(x, Cp, 1), B, 0)
    ga, gb, gc, gd = _sc_gather(xt, nbr_cols)
    return _conv_mm(xt, ga, gb, gc, gd, W, b, relu, C, n)


def _pool(x, nbr_cols, p):
    n = x.shape[0]
    npad = nbr_cols[0].shape[0]
    score = (x @ p) / (jnp.linalg.norm(p) + 1e-12)
    k = int(np.ceil(_RATIO * n))
    kpad = -(-k // 256) * 256
    vals, perm = jax.lax.top_k(score, k)
    perm_pad = _pad_to(perm.astype(jnp.int32), kpad)
    xpp = _sc_gather(_pad_to(x, npad, 0), [perm_pad])
    xp = xpp[:k] * jnp.tanh(vals)[:, None]
    inv = jnp.full((n,), -1, dtype=jnp.int32).at[perm].set(
        jnp.arange(k, dtype=jnp.int32))
    nbp_cols = []
    selfi = jnp.arange(k, dtype=jnp.int32)
    for c in nbr_cols:
        nb = inv[c[perm]]
        nbp = jnp.where(nb < 0, selfi, nb)
        nbp_cols.append(_pad_to(nbp, kpad))
    return xp, nbp_cols, perm, inv


def _unpool(x_small, inv, skip, npad):
    n = skip.shape[0]
    inv_pad = _pad_to(jnp.maximum(inv, 0), npad)
    g = _sc_gather(_pad_to(x_small, -(-x_small.shape[0] // 8) * 8, 0),
                   [inv_pad])
    return skip + jnp.where((inv >= 0)[:, None], g[:n], 0.0)


def kernel(x, edge_index, W_in, b_in, W_d1, b_d1, p1, W_d2, b_d2, p2,
           W_d3, b_d3, p3, W_u1, b_u1, W_u2, b_u2, W_u3, b_u3):
    n0 = x.shape[0]
    np0 = -(-n0 // 256) * 256
    nbr0 = edge_index[1].reshape(-1, 4).astype(jnp.int32)
    nbr0_cols = [_pad_to(nbr0[:, j], np0) for j in range(4)]
    x = _mesh_conv(x, nbr0_cols, W_in, b_in, True)
    down = [(W_d1, b_d1, p1), (W_d2, b_d2, p2), (W_d3, b_d3, p3)]
    up = [(W_u1, b_u1), (W_u2, b_u2), (W_u3, b_u3)]
    skips, invs, res_nbrs, sizes = [], [], [nbr0_cols], []
    nbr_cols = nbr0_cols
    for (W, b, p) in down:
        skips.append(x)
        sizes.append(x.shape[0])
        x, nbr_cols, perm, inv = _pool(x, nbr_cols, p)
        invs.append(inv)
        res_nbrs.append(nbr_cols)
        x = _mesh_conv(x, nbr_cols, W, b, True)
    for j in range(_DEPTH):
        i = _DEPTH - 1 - j
        W, b = up[j]
        npad_i = res_nbrs[i][0].shape[0]
        x = _unpool(x, invs[i], skips[i], npad_i)
        x = _mesh_conv(x, res_nbrs[i], W, b, j < _DEPTH - 1)
    return x


# speedup vs baseline: 1.2038x; 1.0623x over previous
import functools
import numpy as np

import jax
import jax.numpy as jnp
from jax import lax
from jax.experimental import pallas as pl
from jax.experimental.pallas import tpu as pltpu
from jax.experimental.pallas import tpu_sc as plsc

_RATIO = 0.5
_DEPTH = 3
_NC = 2
_NS = 16
_NW = _NC * _NS


def _pad_to(x, m, axis=0):
    n = x.shape[axis]
    p = (-n) % m
    if p == 0:
        return x
    pads = [(0, 0)] * x.ndim
    pads[axis] = (0, p)
    return jnp.pad(x, pads)


def _pick_chunk(b_per_w, d, nbuf=1):
    cap = max(8, (400 * 1024) // (nbuf * d * 4))
    ch = b_per_w
    while ch > cap or ch % 8:
        ch -= 1
        while b_per_w % ch:
            ch -= 1
    return ch


def _sc_gather_spmem_kernel(nidx, nchunks, ch, nbuf, nrows, *refs):
    table = refs[0]
    idxs = refs[1:1 + nidx]
    outs = refs[1 + nidx:1 + 2 * nidx]
    spm = refs[1 + 2 * nidx]
    idx_c = refs[2 + 2 * nidx:2 + 2 * nidx + nbuf]
    bufs = refs[2 + 2 * nidx + nbuf:2 + 2 * nidx + 2 * nbuf]
    isem, gsem, wsem = refs[-3], refs[-2], refs[-1]
    sid = lax.axis_index("s")
    wid = sid * _NC + lax.axis_index("c")
    rch = nrows // _NS
    pltpu.sync_copy(table.at[pl.ds(sid * rch, rch)], spm.at[pl.ds(sid * rch, rch)])
    plsc.subcore_barrier()
    base = wid * (nchunks * ch)
    steps = [(j, c) for j in range(nidx) for c in range(nchunks)]
    T = len(steps)
    G = max(1, nbuf - 2)

    def _idx_dma(t):
        j, c = steps[t]
        return pltpu.async_copy(
            idxs[j].at[pl.ds(base + c * ch, ch)], idx_c[t % nbuf], isem)

    def _gather(t):
        return pltpu.async_copy(
            spm.at[idx_c[t % nbuf]], bufs[t % nbuf], gsem)

    ih, gh, wh = {}, {}, {}
    for t in range(min(G, T)):
        ih[t] = _idx_dma(t)
    for t in range(min(G, T)):
        ih[t].wait()
        gh[t] = _gather(t)
    for t, (j, c) in enumerate(steps):
        gh[t].wait()
        wh[t] = pltpu.async_copy(
            bufs[t % nbuf], outs[j].at[pl.ds(base + c * ch, ch)], wsem)
        nxt = t + G
        if nxt < T:
            if nxt - nbuf >= 0:
                wh[nxt - nbuf].wait()
            ih[nxt] = _idx_dma(nxt)
            ih[nxt].wait()
            gh[nxt] = _gather(nxt)
    for t in range(max(0, T - nbuf), T):
        wh[t].wait()


def _sc_gather_kernel(nidx, nchunks, ch, nbuf, *refs):
    table = refs[0]
    idxs = refs[1:1 + nidx]
    outs = refs[1 + nidx:1 + 2 * nidx]
    idx_c = refs[1 + 2 * nidx:1 + 2 * nidx + nbuf]
    bufs = refs[1 + 2 * nidx + nbuf:1 + 2 * nidx + 2 * nbuf]
    isem, gsem, wsem = refs[-3], refs[-2], refs[-1]
    wid = lax.axis_index("s") * _NC + lax.axis_index("c")
    base = wid * (nchunks * ch)
    steps = [(j, c) for j in range(nidx) for c in range(nchunks)]
    T = len(steps)
    G = max(1, nbuf - 2)

    def _idx_dma(t):
        j, c = steps[t]
        return pltpu.async_copy(
            idxs[j].at[pl.ds(base + c * ch, ch)], idx_c[t % nbuf], isem)

    def _gather(t):
        return pltpu.async_copy(
            table.at[idx_c[t % nbuf]], bufs[t % nbuf], gsem)

    ih, gh, wh = {}, {}, {}
    for t in range(min(G, T)):
        ih[t] = _idx_dma(t)
    for t in range(min(G, T)):
        ih[t].wait()
        gh[t] = _gather(t)
    for t, (j, c) in enumerate(steps):
        gh[t].wait()
        wh[t] = pltpu.async_copy(
            bufs[t % nbuf], outs[j].at[pl.ds(base + c * ch, ch)], wsem)
        nxt = t + G
        if nxt < T:
            if nxt - nbuf >= 0:
                wh[nxt - nbuf].wait()
            ih[nxt] = _idx_dma(nxt)
            ih[nxt].wait()
            gh[nxt] = _gather(nxt)
    for t in range(max(0, T - nbuf), T):
        wh[t].wait()


def _sc_gather(table, idxs):
    nidx = len(idxs)
    B = idxs[0].shape[0]
    T, D = table.shape
    dt = table.dtype
    assert B % _NW == 0, B
    b_per_w = B // _NW
    nbuf = 8
    ch = _pick_chunk(b_per_w, D, nbuf=nbuf)
    nchunks = b_per_w // ch
    mesh = plsc.VectorSubcoreMesh(core_axis_name="c", subcore_axis_name="s",
                                  num_cores=_NC)
    use_spmem = (T % 128 == 0) and (T * D * 4 <= 5000 * 1024)
    if use_spmem:
        body = functools.partial(
            _sc_gather_spmem_kernel, nidx, nchunks, ch, nbuf, T)
        scr = [pltpu.VMEM_SHARED((T, D), dt)]
    else:
        body = functools.partial(_sc_gather_kernel, nidx, nchunks, ch, nbuf)
        scr = []
    kfn = pl.kernel(
        body,
        mesh=mesh,
        out_type=[jax.ShapeDtypeStruct((B, D), dt)] * nidx,
        scratch_types=scr
        + [pltpu.VMEM((ch,), jnp.int32)] * nbuf
        + [pltpu.VMEM((ch, D), dt)] * nbuf
        + [pltpu.SemaphoreType.DMA, pltpu.SemaphoreType.DMA,
           pltpu.SemaphoreType.DMA],
    )
    outs = kfn(table, *idxs)
    return outs if nidx > 1 else outs[0]


def _conv_kernel(x_ref, ga_ref, gb_ref, gc_ref, gd_ref, w_ref, b_ref, o_ref,
                 *, relu):
    xa = ga_ref[...]
    xb = gb_ref[...]
    xc = gc_ref[...]
    xd = gd_ref[...]
    w = w_ref[...]
    acc = jnp.dot(x_ref[...], w[0], preferred_element_type=jnp.float32)
    acc += jnp.dot(jnp.abs(xa - xc), w[1], preferred_element_type=jnp.float32)
    acc += jnp.dot(xa + xc, w[2], preferred_element_type=jnp.float32)
    acc += jnp.dot(jnp.abs(xb - xd), w[3], preferred_element_type=jnp.float32)
    acc += jnp.dot(xb + xd, w[4], preferred_element_type=jnp.float32)
    acc = acc + b_ref[...]
    if relu:
        acc = jnp.maximum(acc, 0.0)
    o_ref[...] = acc


def _conv_mm(xt, ga, gb, gc, gd, W, b, relu, C, n):
    B, Cp = xt.shape
    H = W.shape[1]
    BN = 512 if B % 512 == 0 else 256
    Wb = jnp.pad(W.reshape(5, C, H), ((0, 0), (0, Cp - C), (0, 0)))
    grid = (B // BN,)
    out = pl.pallas_call(
        functools.partial(_conv_kernel, relu=relu),
        grid=grid,
        in_specs=[pl.BlockSpec((BN, Cp), lambda i: (i, 0))] * 5
        + [pl.BlockSpec((5, Cp, H), lambda i: (0, 0, 0)),
           pl.BlockSpec((1, H), lambda i: (0, 0))],
        out_specs=pl.BlockSpec((BN, H), lambda i: (i, 0)),
        out_shape=jax.ShapeDtypeStruct((B, H), jnp.float32),
    )(xt, ga, gb, gc, gd, Wb, b.reshape(1, H))
    return out[:n]


def _mesh_conv(x, nbr_cols, W, b, relu):
    n, C = x.shape
    Cp = 128
    B = nbr_cols[0].shape[0]
    xt = _pad_to(_pad_to(x, Cp, 1), B, 0)
    ga, gb, gc, gd = _sc_gather(xt, nbr_cols)
    return _conv_mm(xt, ga, gb, gc, gd, W, b, relu, C, n)


def _pool(x, nbr_cols, p):
    n = x.shape[0]
    npad = nbr_cols[0].shape[0]
    score = (x @ p) / (jnp.linalg.norm(p) + 1e-12)
    k = int(np.ceil(_RATIO * n))
    kpad = -(-k // 256) * 256
    vals, perm = jax.lax.top_k(score, k)
    perm_pad = _pad_to(perm.astype(jnp.int32), kpad)
    xpp = _sc_gather(_pad_to(x, npad, 0), [perm_pad])
    xp = xpp[:k] * jnp.tanh(vals)[:, None]
    inv = jnp.full((n,), -1, dtype=jnp.int32).at[perm].set(
        jnp.arange(k, dtype=jnp.int32))
    nbp_cols = []
    selfi = jnp.arange(k, dtype=jnp.int32)
    for c in nbr_cols:
        nb = inv[c[perm]]
        nbp = jnp.where(nb < 0, selfi, nb)
        nbp_cols.append(_pad_to(nbp, kpad))
    return xp, nbp_cols, perm, inv


def _unpool(x_small, inv, skip, npad):
    n = skip.shape[0]
    inv_pad = _pad_to(jnp.maximum(inv, 0), npad)
    g = _sc_gather(_pad_to(x_small, -(-x_small.shape[0] // 128) * 128, 0),
                   [inv_pad])
    return skip + jnp.where((inv >= 0)[:, None], g[:n], 0.0)


def kernel(x, edge_index, W_in, b_in, W_d1, b_d1, p1, W_d2, b_d2, p2,
           W_d3, b_d3, p3, W_u1, b_u1, W_u2, b_u2, W_u3, b_u3):
    n0 = x.shape[0]
    np0 = -(-n0 // 256) * 256
    nbr0 = edge_index[1].reshape(-1, 4).astype(jnp.int32)
    nbr0_cols = [_pad_to(nbr0[:, j], np0) for j in range(4)]
    x = _mesh_conv(x, nbr0_cols, W_in, b_in, True)
    down = [(W_d1, b_d1, p1), (W_d2, b_d2, p2), (W_d3, b_d3, p3)]
    up = [(W_u1, b_u1), (W_u2, b_u2), (W_u3, b_u3)]
    skips, invs, res_nbrs, sizes = [], [], [nbr0_cols], []
    nbr_cols = nbr0_cols
    for (W, b, p) in down:
        skips.append(x)
        sizes.append(x.shape[0])
        x, nbr_cols, perm, inv = _pool(x, nbr_cols, p)
        invs.append(inv)
        res_nbrs.append(nbr_cols)
        x = _mesh_conv(x, nbr_cols, W, b, True)
    for j in range(_DEPTH):
        i = _DEPTH - 1 - j
        W, b = up[j]
        npad_i = res_nbrs[i][0].shape[0]
        x = _unpool(x, invs[i], skips[i], npad_i)
        x = _mesh_conv(x, res_nbrs[i], W, b, j < _DEPTH - 1)
    return x
